# baseline (speedup 1.0000x reference)
import jax
import jax.numpy as jnp
from jax import lax
from jax.experimental import pallas as pl
from jax.experimental.pallas import tpu as pltpu
from jax.experimental.pallas import tpu_sc as plsc

H = 512
W = 512
N = H * W
NC = 2
NS = 16
L = 16
NW = NC * NS
NB = NW
TPW = N // NW
SPW = N // NW
CH = 8192
NBIN = N + NB * L + CH


def _round_half_even_nonneg(x):
    n = x.astype(jnp.int32)
    f = x - n.astype(jnp.float32)
    half = jnp.full((L,), 0.5, jnp.float32)
    up = (f > half) | ((f == half) & ((n & 1) == 1))
    return jnp.where(up, n + 1, n)


def _flat_vreg(bx, by, base, j):
    off = j * L
    lin = base + off + lax.iota(jnp.int32, L)
    xi = lin & (W - 1)
    yi = lax.shift_right_logical(lin, 9)
    px = xi.astype(jnp.float32) + bx[pl.ds(off, L)]
    py = yi.astype(jnp.float32) + by[pl.ds(off, L)]
    px = jnp.minimum(jnp.maximum(px, 0.0), float(W - 1))
    py = jnp.minimum(jnp.maximum(py, 0.0), float(H - 1))
    tx = _round_half_even_nonneg(px)
    ty = _round_half_even_nonneg(py)
    return lax.shift_left(ty, 9) | tx


def _any_f32(p):
    return jnp.sum(jnp.where(p, jnp.full((L,), 1.0, jnp.float32),
                             jnp.zeros((L,), jnp.float32)))


def _hist_body(fx_hbm, fy_hbm, hist_hbm, bx, by, cnt32, sem):
    wid = lax.axis_index("s") * NC + lax.axis_index("c")
    base = wid * SPW
    cx = pltpu.async_copy(fx_hbm.at[pl.ds(base, SPW)], bx, sem)
    cy = pltpu.async_copy(fy_hbm.at[pl.ds(base, SPW)], by, sem)
    cx.wait()
    cy.wait()

    cnt32[pl.ds(0, L)] = jnp.zeros((L,), jnp.int32)
    cnt32[pl.ds(L, L)] = jnp.zeros((L,), jnp.int32)

    @pl.loop(0, SPW // L)
    def _per_vreg(j):
        fl = _flat_vreg(bx, by, base, j)
        o = lax.shift_right_logical(fl, 13)
        c, last = plsc.scan_count(o)
        cur = plsc.load_gather(cnt32, [o])
        plsc.store_scatter(cnt32, [o], cur + c, mask=last)

    pltpu.sync_copy(cnt32, hist_hbm.at[pl.ds(wid * NB, NB)])


def _bucket_layout(bh, wid):
    zero_i = jnp.zeros((L,), jnp.int32)
    tot0 = zero_i
    tot1 = zero_i
    pre0 = zero_i
    pre1 = zero_i

    def _acc(wi, carry):
        t0, t1, p0, p1 = carry
        row0 = bh[pl.ds(wi * NB, L)]
        row1 = bh[pl.ds(wi * NB + L, L)]
        sel = jnp.full((L,), jnp.where(wi < wid, 1, 0), jnp.int32)
        return (t0 + row0, t1 + row1, p0 + row0 * sel, p1 + row1 * sel)

    tot0, tot1, pre0, pre1 = lax.fori_loop(
        0, NW, _acc, (tot0, tot1, pre0, pre1))

    pad0 = (tot0 + (L - 1)) & ~(L - 1)
    pad1 = (tot1 + (L - 1)) & ~(L - 1)
    excl0 = plsc.cumsum(pad0) - pad0
    carry = jnp.sum(pad0)
    excl1 = plsc.cumsum(pad1) - pad1 + carry
    return excl0, excl1, tot0, tot1, pre0, pre1


def _permute_body(fx_hbm, fy_hbm, hist_hbm, d_hbm, i0_hbm, i1_hbm, i2_hbm,
                  fb_hbm, db_hbm, v0_hbm, v1_hbm, v2_hbm,
                  bx, by, bh, cnt32, bd, b0, b1, b2, bfl, didx, sem):
    wid = lax.axis_index("s") * NC + lax.axis_index("c")
    base = wid * SPW
    ch = pltpu.async_copy(hist_hbm, bh, sem)
    cx = pltpu.async_copy(fx_hbm.at[pl.ds(base, SPW)], bx, sem)
    cy = pltpu.async_copy(fy_hbm.at[pl.ds(base, SPW)], by, sem)
    cd = pltpu.async_copy(d_hbm.at[pl.ds(base, SPW)], bd, sem)
    c0 = pltpu.async_copy(i0_hbm.at[pl.ds(base, SPW)], b0, sem)
    c1 = pltpu.async_copy(i1_hbm.at[pl.ds(base, SPW)], b1, sem)
    c2 = pltpu.async_copy(i2_hbm.at[pl.ds(base, SPW)], b2, sem)
    ch.wait()

    excl0, excl1, _, _, pre0, pre1 = _bucket_layout(bh, wid)
    cnt32[pl.ds(0, L)] = excl0 + pre0
    cnt32[pl.ds(L, L)] = excl1 + pre1

    cx.wait()
    cy.wait()

    @pl.loop(0, SPW // L)
    def _per_vreg(j):
        off = j * L
        fl = _flat_vreg(bx, by, base, j)
        bfl[pl.ds(off, L)] = fl
        o = lax.shift_right_logical(fl, 13)
        c, last = plsc.scan_count(o)
        cur = plsc.load_gather(cnt32, [o])
        pos = cur + c - 1
        didx[pl.ds(off, L)] = pos
        plsc.store_scatter(cnt32, [o], cur + c, mask=last)

    cd.wait()
    c0.wait()
    c1.wait()
    c2.wait()

    sf = pltpu.async_copy(bfl, fb_hbm.at[didx], sem)
    sd = pltpu.async_copy(bd, db_hbm.at[didx], sem)
    s0 = pltpu.async_copy(b0, v0_hbm.at[didx], sem)
    s1 = pltpu.async_copy(b1, v1_hbm.at[didx], sem)
    s2 = pltpu.async_copy(b2, v2_hbm.at[didx], sem)
    sf.wait()
    sd.wait()
    s0.wait()
    s1.wait()
    s2.wait()


def _warp_body(hist_hbm, fb_hbm, db_hbm, v0_hbm, v1_hbm, v2_hbm, sr_hbm,
               o0_hbm, o1_hbm, o2_hbm,
               zbuf, acc0, acc1, acc2, cnt, bh, bf, bd, b0, b1, b2, bsr,
               sem):
    wid = lax.axis_index("s") * NC + lax.axis_index("c")
    tbase = wid * TPW
    ch = pltpu.async_copy(hist_hbm, bh, sem)
    pltpu.sync_copy(sr_hbm, bsr)
    srv = bsr[...]

    big = jnp.full((L,), 1e30, jnp.float32)
    zero = jnp.zeros((L,), jnp.float32)

    @pl.loop(0, TPW // L)
    def _init(i):
        off = i * L
        zbuf[pl.ds(off, L)] = big
        acc0[pl.ds(off, L)] = zero
        acc1[pl.ds(off, L)] = zero
        acc2[pl.ds(off, L)] = zero
        cnt[pl.ds(off, L)] = zero

    ch.wait()
    excl0, excl1, tot0, tot1, _, _ = _bucket_layout(bh, wid)

    iot = lax.iota(jnp.int32, L)
    lane = jnp.where(iot == jnp.full((L,), wid & (L - 1)),
                     jnp.full((L,), 1, jnp.int32), jnp.zeros((L,), jnp.int32))
    in_hi = lax.shift_right_logical(wid, 4)
    rstart = jnp.sum(excl0 * lane) * (1 - in_hi) + jnp.sum(excl1 * lane) * in_hi
    rstart = pl.multiple_of(rstart, L)
    rcnt = jnp.sum(tot0 * lane) * (1 - in_hi) + jnp.sum(tot1 * lane) * in_hi
    nch = (rcnt + CH - 1) // CH

    @pl.loop(0, nch)
    def _p1(c):
        cbase = rstart + c * CH
        cf = pltpu.async_copy(fb_hbm.at[pl.ds(cbase, CH)], bf, sem)
        cd = pltpu.async_copy(db_hbm.at[pl.ds(cbase, CH)], bd, sem)
        cf.wait()
        cd.wait()
        done = c * CH

        @pl.loop(0, CH // L)
        def _vreg(j):
            off = j * L
            valid = (done + off + iot) < rcnt
            fl = bf[pl.ds(off, L)]
            dd = bd[pl.ds(off, L)]
            ridx = (fl - tbase) & (TPW - 1)

            def _body(_):
                cur = plsc.load_gather(zbuf, [ridx], mask=valid)
                pend = valid & (dd < cur)
                plsc.store_scatter(zbuf, [ridx], dd, mask=pend)
                cur2 = plsc.load_gather(zbuf, [ridx], mask=valid)
                return _any_f32(valid & (dd < cur2))

            lax.while_loop(lambda t: t > 0.0, _body, _any_f32(valid))

    @pl.loop(0, nch)
    def _p2(c):
        cbase = rstart + c * CH
        cf = pltpu.async_copy(fb_hbm.at[pl.ds(cbase, CH)], bf, sem)
        cd = pltpu.async_copy(db_hbm.at[pl.ds(cbase, CH)], bd, sem)
        c0 = pltpu.async_copy(v0_hbm.at[pl.ds(cbase, CH)], b0, sem)
        c1 = pltpu.async_copy(v1_hbm.at[pl.ds(cbase, CH)], b1, sem)
        c2 = pltpu.async_copy(v2_hbm.at[pl.ds(cbase, CH)], b2, sem)
        cf.wait()
        cd.wait()
        c0.wait()
        c1.wait()
        c2.wait()
        done = c * CH

        @pl.loop(0, CH // L)
        def _vreg(j):
            off = j * L
            valid = (done + off + iot) < rcnt
            fl = bf[pl.ds(off, L)]
            dd = bd[pl.ds(off, L)]
            ridx = (fl - tbase) & (TPW - 1)
            zm = plsc.load_gather(zbuf, [ridx], mask=valid)
            ok = valid & (dd <= zm + srv)
            one = jnp.where(ok, jnp.full((L,), 1.0, jnp.float32), zero)
            plsc.addupdate_scatter(cnt, [ridx], one, mask=ok)
            plsc.addupdate_scatter(acc0, [ridx], b0[pl.ds(off, L)], mask=ok)
            plsc.addupdate_scatter(acc1, [ridx], b1[pl.ds(off, L)], mask=ok)
            plsc.addupdate_scatter(acc2, [ridx], b2[pl.ds(off, L)], mask=ok)

    @pl.loop(0, TPW // L)
    def _fin(i):
        off = i * L
        inv = 1.0 / jnp.maximum(cnt[pl.ds(off, L)], 1.0)
        acc0[pl.ds(off, L)] = acc0[pl.ds(off, L)] * inv
        acc1[pl.ds(off, L)] = acc1[pl.ds(off, L)] * inv
        acc2[pl.ds(off, L)] = acc2[pl.ds(off, L)] * inv

    pltpu.sync_copy(acc0, o0_hbm.at[pl.ds(tbase, TPW)])
    pltpu.sync_copy(acc1, o1_hbm.at[pl.ds(tbase, TPW)])
    pltpu.sync_copy(acc2, o2_hbm.at[pl.ds(tbase, TPW)])


@jax.jit
def _run(fx, fy, d, i0, i1, i2, srv):
    mesh = plsc.VectorSubcoreMesh(core_axis_name="c", subcore_axis_name="s")
    params = pltpu.CompilerParams(needs_layout_passes=False)

    hist = pl.kernel(
        _hist_body,
        out_type=jax.ShapeDtypeStruct((NW * NB,), jnp.int32),
        mesh=mesh,
        compiler_params=params,
        scratch_types=[
            pltpu.VMEM((SPW,), jnp.float32),
            pltpu.VMEM((SPW,), jnp.float32),
            pltpu.VMEM((NB,), jnp.int32),
            pltpu.SemaphoreType.DMA,
        ],
    )(fx, fy)

    fb, db, v0, v1, v2 = pl.kernel(
        _permute_body,
        out_type=(
            jax.ShapeDtypeStruct((NBIN,), jnp.int32),
            jax.ShapeDtypeStruct((NBIN,), jnp.float32),
            jax.ShapeDtypeStruct((NBIN,), jnp.float32),
            jax.ShapeDtypeStruct((NBIN,), jnp.float32),
            jax.ShapeDtypeStruct((NBIN,), jnp.float32),
        ),
        mesh=mesh,
        compiler_params=params,
        scratch_types=[
            pltpu.VMEM((SPW,), jnp.float32),
            pltpu.VMEM((SPW,), jnp.float32),
            pltpu.VMEM((NW * NB,), jnp.int32),
            pltpu.VMEM((NB,), jnp.int32),
            pltpu.VMEM((SPW,), jnp.float32),
            pltpu.VMEM((SPW,), jnp.float32),
            pltpu.VMEM((SPW,), jnp.float32),
            pltpu.VMEM((SPW,), jnp.float32),
            pltpu.VMEM((SPW,), jnp.int32),
            pltpu.VMEM((SPW,), jnp.int32),
            pltpu.SemaphoreType.DMA,
        ],
    )(fx, fy, hist, d, i0, i1, i2)

    o0, o1, o2 = pl.kernel(
        _warp_body,
        out_type=(
            jax.ShapeDtypeStruct((N,), jnp.float32),
            jax.ShapeDtypeStruct((N,), jnp.float32),
            jax.ShapeDtypeStruct((N,), jnp.float32),
        ),
        mesh=mesh,
        compiler_params=params,
        scratch_types=[
            pltpu.VMEM((TPW,), jnp.float32),
            pltpu.VMEM((TPW,), jnp.float32),
            pltpu.VMEM((TPW,), jnp.float32),
            pltpu.VMEM((TPW,), jnp.float32),
            pltpu.VMEM((TPW,), jnp.float32),
            pltpu.VMEM((NW * NB,), jnp.int32),
            pltpu.VMEM((CH,), jnp.int32),
            pltpu.VMEM((CH,), jnp.float32),
            pltpu.VMEM((CH,), jnp.float32),
            pltpu.VMEM((CH,), jnp.float32),
            pltpu.VMEM((CH,), jnp.float32),
            pltpu.VMEM((L,), jnp.float32),
            pltpu.SemaphoreType.DMA,
        ],
    )(hist, fb, db, v0, v1, v2, srv)

    out = jnp.stack([o0, o1, o2], axis=-1)
    return out.reshape(H, W, 3)


def kernel(img, flow, depth, same_range):
    fx = flow[0, :, :, 0].reshape(-1)
    fy = flow[0, :, :, 1].reshape(-1)
    d = depth.reshape(-1)
    i0 = img[:, :, 0].reshape(-1)
    i1 = img[:, :, 1].reshape(-1)
    i2 = img[:, :, 2].reshape(-1)
    srv = jnp.full((L,), same_range, jnp.float32)
    return _run(fx, fy, d, i0, i1, i2, srv)


# speedup vs baseline: 1.0003x; 1.0003x over previous
import jax
import jax.numpy as jnp
from jax import lax
from jax.experimental import pallas as pl
from jax.experimental.pallas import tpu as pltpu
from jax.experimental.pallas import tpu_sc as plsc

H = 512
W = 512
N = H * W
NC = 2
NS = 16
L = 16
NW = NC * NS
NB = NW
TPW = N // NW
SPW = N // NW
CH = 8192
NBIN = N + NB * L + CH


def _round_half_even_nonneg(x):
    n = x.astype(jnp.int32)
    f = x - n.astype(jnp.float32)
    half = jnp.full((L,), 0.5, jnp.float32)
    up = (f > half) | ((f == half) & ((n & 1) == 1))
    return jnp.where(up, n + 1, n)


def _flat_vreg(bx, by, base, j):
    off = j * L
    lin = base + off + lax.iota(jnp.int32, L)
    xi = lin & (W - 1)
    yi = lax.shift_right_logical(lin, 9)
    px = xi.astype(jnp.float32) + bx[pl.ds(off, L)]
    py = yi.astype(jnp.float32) + by[pl.ds(off, L)]
    px = jnp.minimum(jnp.maximum(px, 0.0), float(W - 1))
    py = jnp.minimum(jnp.maximum(py, 0.0), float(H - 1))
    tx = _round_half_even_nonneg(px)
    ty = _round_half_even_nonneg(py)
    return lax.shift_left(ty, 9) | tx


def _any_f32(p):
    return jnp.sum(jnp.where(p, jnp.full((L,), 1.0, jnp.float32),
                             jnp.zeros((L,), jnp.float32)))


def _hist_body(fx_hbm, fy_hbm, hist_hbm, bx, by, cnt32, sem):
    wid = lax.axis_index("s") * NC + lax.axis_index("c")
    base = wid * SPW
    cx = pltpu.async_copy(fx_hbm.at[pl.ds(base, SPW)], bx, sem)
    cy = pltpu.async_copy(fy_hbm.at[pl.ds(base, SPW)], by, sem)
    cx.wait()
    cy.wait()

    cnt32[pl.ds(0, L)] = jnp.zeros((L,), jnp.int32)
    cnt32[pl.ds(L, L)] = jnp.zeros((L,), jnp.int32)

    @pl.loop(0, SPW // L)
    def _per_vreg(j):
        fl = _flat_vreg(bx, by, base, j)
        o = lax.shift_right_logical(fl, 13)
        c, last = plsc.scan_count(o)
        cur = plsc.load_gather(cnt32, [o])
        plsc.store_scatter(cnt32, [o], cur + c, mask=last)

    pltpu.sync_copy(cnt32, hist_hbm.at[pl.ds(wid * NB, NB)])


def _bucket_layout(bh, wid):
    zero_i = jnp.zeros((L,), jnp.int32)
    tot0 = zero_i
    tot1 = zero_i
    pre0 = zero_i
    pre1 = zero_i

    def _acc(wi, carry):
        t0, t1, p0, p1 = carry
        row0 = bh[pl.ds(wi * NB, L)]
        row1 = bh[pl.ds(wi * NB + L, L)]
        sel = jnp.full((L,), jnp.where(wi < wid, 1, 0), jnp.int32)
        return (t0 + row0, t1 + row1, p0 + row0 * sel, p1 + row1 * sel)

    tot0, tot1, pre0, pre1 = lax.fori_loop(
        0, NW, _acc, (tot0, tot1, pre0, pre1))

    pad0 = (tot0 + (L - 1)) & ~(L - 1)
    pad1 = (tot1 + (L - 1)) & ~(L - 1)
    excl0 = plsc.cumsum(pad0) - pad0
    carry = jnp.sum(pad0)
    excl1 = plsc.cumsum(pad1) - pad1 + carry
    return excl0, excl1, tot0, tot1, pre0, pre1


def _permute_body(fx_hbm, fy_hbm, hist_hbm, d_hbm, i0_hbm, i1_hbm, i2_hbm,
                  fb_hbm, db_hbm, v0_hbm, v1_hbm, v2_hbm,
                  bx, by, bh, cnt32, bd, b0, b1, b2, bfl, didx, sem):
    wid = lax.axis_index("s") * NC + lax.axis_index("c")
    base = wid * SPW
    ch = pltpu.async_copy(hist_hbm, bh, sem)
    cx = pltpu.async_copy(fx_hbm.at[pl.ds(base, SPW)], bx, sem)
    cy = pltpu.async_copy(fy_hbm.at[pl.ds(base, SPW)], by, sem)
    cd = pltpu.async_copy(d_hbm.at[pl.ds(base, SPW)], bd, sem)
    c0 = pltpu.async_copy(i0_hbm.at[pl.ds(base, SPW)], b0, sem)
    c1 = pltpu.async_copy(i1_hbm.at[pl.ds(base, SPW)], b1, sem)
    c2 = pltpu.async_copy(i2_hbm.at[pl.ds(base, SPW)], b2, sem)
    ch.wait()

    excl0, excl1, _, _, pre0, pre1 = _bucket_layout(bh, wid)
    cnt32[pl.ds(0, L)] = excl0 + pre0
    cnt32[pl.ds(L, L)] = excl1 + pre1

    cx.wait()
    cy.wait()

    @pl.loop(0, SPW // L)
    def _per_vreg(j):
        off = j * L
        fl = _flat_vreg(bx, by, base, j)
        bfl[pl.ds(off, L)] = fl
        o = lax.shift_right_logical(fl, 13)
        c, last = plsc.scan_count(o)
        cur = plsc.load_gather(cnt32, [o])
        pos = cur + c - 1
        didx[j // 8, pl.ds((j % 8) * L, L)] = pos
        plsc.store_scatter(cnt32, [o], cur + c, mask=last)

    cd.wait()
    c0.wait()
    c1.wait()
    c2.wait()

    @pl.loop(0, SPW // 128)
    def _scat(r):
        off = r * 128
        sf = pltpu.async_copy(bfl.at[pl.ds(off, 128)], fb_hbm.at[didx.at[r]], sem)
        sd = pltpu.async_copy(bd.at[pl.ds(off, 128)], db_hbm.at[didx.at[r]], sem)
        s0 = pltpu.async_copy(b0.at[pl.ds(off, 128)], v0_hbm.at[didx.at[r]], sem)
        s1 = pltpu.async_copy(b1.at[pl.ds(off, 128)], v1_hbm.at[didx.at[r]], sem)
        s2 = pltpu.async_copy(b2.at[pl.ds(off, 128)], v2_hbm.at[didx.at[r]], sem)
        sf.wait()
        sd.wait()
        s0.wait()
        s1.wait()
        s2.wait()


def _warp_body(hist_hbm, fb_hbm, db_hbm, v0_hbm, v1_hbm, v2_hbm, sr_hbm,
               o0_hbm, o1_hbm, o2_hbm,
               zbuf, acc0, acc1, acc2, cnt, bh, bf, bd, b0, b1, b2, bsr,
               sem):
    wid = lax.axis_index("s") * NC + lax.axis_index("c")
    tbase = wid * TPW
    ch = pltpu.async_copy(hist_hbm, bh, sem)
    pltpu.sync_copy(sr_hbm, bsr)
    srv = bsr[...]

    big = jnp.full((L,), 1e30, jnp.float32)
    zero = jnp.zeros((L,), jnp.float32)

    @pl.loop(0, TPW // L)
    def _init(i):
        off = i * L
        zbuf[pl.ds(off, L)] = big
        acc0[pl.ds(off, L)] = zero
        acc1[pl.ds(off, L)] = zero
        acc2[pl.ds(off, L)] = zero
        cnt[pl.ds(off, L)] = zero

    ch.wait()
    excl0, excl1, tot0, tot1, _, _ = _bucket_layout(bh, wid)

    iot = lax.iota(jnp.int32, L)
    lane = jnp.where(iot == jnp.full((L,), wid & (L - 1)),
                     jnp.full((L,), 1, jnp.int32), jnp.zeros((L,), jnp.int32))
    in_hi = lax.shift_right_logical(wid, 4)
    rstart = jnp.sum(excl0 * lane) * (1 - in_hi) + jnp.sum(excl1 * lane) * in_hi
    rstart = pl.multiple_of(rstart, L)
    rcnt = jnp.sum(tot0 * lane) * (1 - in_hi) + jnp.sum(tot1 * lane) * in_hi
    nch = (rcnt + CH - 1) // CH

    @pl.loop(0, nch)
    def _p1(c):
        cbase = rstart + c * CH
        cf = pltpu.async_copy(fb_hbm.at[pl.ds(cbase, CH)], bf, sem)
        cd = pltpu.async_copy(db_hbm.at[pl.ds(cbase, CH)], bd, sem)
        cf.wait()
        cd.wait()
        done = c * CH

        @pl.loop(0, CH // L)
        def _vreg(j):
            off = j * L
            valid = (done + off + iot) < rcnt
            fl = bf[pl.ds(off, L)]
            dd = bd[pl.ds(off, L)]
            ridx = (fl - tbase) & (TPW - 1)

            def _body(_):
                cur = plsc.load_gather(zbuf, [ridx], mask=valid)
                pend = valid & (dd < cur)
                plsc.store_scatter(zbuf, [ridx], dd, mask=pend)
                cur2 = plsc.load_gather(zbuf, [ridx], mask=valid)
                return _any_f32(valid & (dd < cur2))

            lax.while_loop(lambda t: t > 0.0, _body, _any_f32(valid))

    @pl.loop(0, nch)
    def _p2(c):
        cbase = rstart + c * CH
        cf = pltpu.async_copy(fb_hbm.at[pl.ds(cbase, CH)], bf, sem)
        cd = pltpu.async_copy(db_hbm.at[pl.ds(cbase, CH)], bd, sem)
        c0 = pltpu.async_copy(v0_hbm.at[pl.ds(cbase, CH)], b0, sem)
        c1 = pltpu.async_copy(v1_hbm.at[pl.ds(cbase, CH)], b1, sem)
        c2 = pltpu.async_copy(v2_hbm.at[pl.ds(cbase, CH)], b2, sem)
        cf.wait()
        cd.wait()
        c0.wait()
        c1.wait()
        c2.wait()
        done = c * CH

        @pl.loop(0, CH // L)
        def _vreg(j):
            off = j * L
            valid = (done + off + iot) < rcnt
            fl = bf[pl.ds(off, L)]
            dd = bd[pl.ds(off, L)]
            ridx = (fl - tbase) & (TPW - 1)
            zm = plsc.load_gather(zbuf, [ridx], mask=valid)
            ok = valid & (dd <= zm + srv)
            one = jnp.where(ok, jnp.full((L,), 1.0, jnp.float32), zero)
            plsc.addupdate_scatter(cnt, [ridx], one, mask=ok)
            plsc.addupdate_scatter(acc0, [ridx], b0[pl.ds(off, L)], mask=ok)
            plsc.addupdate_scatter(acc1, [ridx], b1[pl.ds(off, L)], mask=ok)
            plsc.addupdate_scatter(acc2, [ridx], b2[pl.ds(off, L)], mask=ok)

    @pl.loop(0, TPW // L)
    def _fin(i):
        off = i * L
        inv = 1.0 / jnp.maximum(cnt[pl.ds(off, L)], 1.0)
        acc0[pl.ds(off, L)] = acc0[pl.ds(off, L)] * inv
        acc1[pl.ds(off, L)] = acc1[pl.ds(off, L)] * inv
        acc2[pl.ds(off, L)] = acc2[pl.ds(off, L)] * inv

    pltpu.sync_copy(acc0, o0_hbm.at[pl.ds(tbase, TPW)])
    pltpu.sync_copy(acc1, o1_hbm.at[pl.ds(tbase, TPW)])
    pltpu.sync_copy(acc2, o2_hbm.at[pl.ds(tbase, TPW)])


@jax.jit
def _run(fx, fy, d, i0, i1, i2, srv):
    mesh = plsc.VectorSubcoreMesh(core_axis_name="c", subcore_axis_name="s")
    params = pltpu.CompilerParams(needs_layout_passes=False)

    hist = pl.kernel(
        _hist_body,
        out_type=jax.ShapeDtypeStruct((NW * NB,), jnp.int32),
        mesh=mesh,
        compiler_params=params,
        scratch_types=[
            pltpu.VMEM((SPW,), jnp.float32),
            pltpu.VMEM((SPW,), jnp.float32),
            pltpu.VMEM((NB,), jnp.int32),
            pltpu.SemaphoreType.DMA,
        ],
    )(fx, fy)

    fb, db, v0, v1, v2 = pl.kernel(
        _permute_body,
        out_type=(
            jax.ShapeDtypeStruct((NBIN,), jnp.int32),
            jax.ShapeDtypeStruct((NBIN,), jnp.float32),
            jax.ShapeDtypeStruct((NBIN,), jnp.float32),
            jax.ShapeDtypeStruct((NBIN,), jnp.float32),
            jax.ShapeDtypeStruct((NBIN,), jnp.float32),
        ),
        mesh=mesh,
        compiler_params=params,
        scratch_types=[
            pltpu.VMEM((SPW,), jnp.float32),
            pltpu.VMEM((SPW,), jnp.float32),
            pltpu.VMEM((NW * NB,), jnp.int32),
            pltpu.VMEM((NB,), jnp.int32),
            pltpu.VMEM((SPW,), jnp.float32),
            pltpu.VMEM((SPW,), jnp.float32),
            pltpu.VMEM((SPW,), jnp.float32),
            pltpu.VMEM((SPW,), jnp.float32),
            pltpu.VMEM((SPW,), jnp.int32),
            pltpu.VMEM((SPW // 128, 128), jnp.int32),
            pltpu.SemaphoreType.DMA,
        ],
    )(fx, fy, hist, d, i0, i1, i2)

    o0, o1, o2 = pl.kernel(
        _warp_body,
        out_type=(
            jax.ShapeDtypeStruct((N,), jnp.float32),
            jax.ShapeDtypeStruct((N,), jnp.float32),
            jax.ShapeDtypeStruct((N,), jnp.float32),
        ),
        mesh=mesh,
        compiler_params=params,
        scratch_types=[
            pltpu.VMEM((TPW,), jnp.float32),
            pltpu.VMEM((TPW,), jnp.float32),
            pltpu.VMEM((TPW,), jnp.float32),
            pltpu.VMEM((TPW,), jnp.float32),
            pltpu.VMEM((TPW,), jnp.float32),
            pltpu.VMEM((NW * NB,), jnp.int32),
            pltpu.VMEM((CH,), jnp.int32),
            pltpu.VMEM((CH,), jnp.float32),
            pltpu.VMEM((CH,), jnp.float32),
            pltpu.VMEM((CH,), jnp.float32),
            pltpu.VMEM((CH,), jnp.float32),
            pltpu.VMEM((L,), jnp.float32),
            pltpu.SemaphoreType.DMA,
        ],
    )(hist, fb, db, v0, v1, v2, srv)

    out = jnp.stack([o0, o1, o2], axis=-1)
    return out.reshape(H, W, 3)


def kernel(img, flow, depth, same_range):
    fx = flow[0, :, :, 0].reshape(-1)
    fy = flow[0, :, :, 1].reshape(-1)
    d = depth.reshape(-1)
    i0 = img[:, :, 0].reshape(-1)
    i1 = img[:, :, 1].reshape(-1)
    i2 = img[:, :, 2].reshape(-1)
    srv = jnp.full((L,), same_range, jnp.float32)
    return _run(fx, fy, d, i0, i1, i2, srv)


# speedup vs baseline: 12.9856x; 12.9817x over previous
import jax
import jax.numpy as jnp
from jax import lax
from jax.experimental import pallas as pl
from jax.experimental.pallas import tpu as pltpu
from jax.experimental.pallas import tpu_sc as plsc

H = 512
W = 512
N = H * W
NC = 2
NS = 16
L = 16
NW = NC * NS
SPW = N // NW
TPB = N // NS
CH = 8192
NSRC = N // NC
NBIN = NSRC + NS * L + CH
STW = NBIN // NS


def _round_half_even_nonneg(x):
    n = x.astype(jnp.int32)
    f = x - n.astype(jnp.float32)
    half = jnp.full((L,), 0.5, jnp.float32)
    up = (f > half) | ((f == half) & ((n & 1) == 1))
    return jnp.where(up, n + 1, n)


def _flat_vreg(bx, by, base, j):
    off = j * L
    lin = base + off + lax.iota(jnp.int32, L)
    xi = lin & (W - 1)
    yi = lax.shift_right_logical(lin, 9)
    px = xi.astype(jnp.float32) + bx[pl.ds(off, L)]
    py = yi.astype(jnp.float32) + by[pl.ds(off, L)]
    px = jnp.minimum(jnp.maximum(px, 0.0), float(W - 1))
    py = jnp.minimum(jnp.maximum(py, 0.0), float(H - 1))
    tx = _round_half_even_nonneg(px)
    ty = _round_half_even_nonneg(py)
    return lax.shift_left(ty, 9) | tx


def _any_f32(p):
    return jnp.sum(jnp.where(p, jnp.full((L,), 1.0, jnp.float32),
                             jnp.zeros((L,), jnp.float32)))


def _bucket_layout(bh, sid):
    zero_i = jnp.zeros((L,), jnp.int32)

    def _acc(wi, carry):
        t, p = carry
        row = bh[pl.ds(wi * NS, L)]
        sel = jnp.full((L,), jnp.where(wi < sid, 1, 0), jnp.int32)
        return (t + row, p + row * sel)

    tot, pre = lax.fori_loop(0, NS, _acc, (zero_i, zero_i))
    padt = (tot + (L - 1)) & ~(L - 1)
    excl = plsc.cumsum(padt) - padt
    return excl, tot, pre, padt


def _lane_scalar(vec, sid):
    iot = lax.iota(jnp.int32, L)
    lane = jnp.where(iot == jnp.full((L,), sid),
                     jnp.full((L,), 1, jnp.int32), jnp.zeros((L,), jnp.int32))
    return jnp.sum(vec * lane)


def _bin_body(fx_hbm, fy_hbm, d_hbm, i0_hbm, i1_hbm, i2_hbm,
              pz_hbm, histo_hbm, fb_hbm, db_hbm, v0_hbm, v1_hbm, v2_hbm,
              bfl, bx, by, bdp, b2p, didx, bf, zbuf, bh, cnt16, padv,
              sfb, sdb, s0b, s1b, s2b, histS, sem):
    c = lax.axis_index("c")
    sid = lax.axis_index("s")
    wid = sid * NC + c
    sbase = wid * SPW
    tbase = sid * TPB

    cx = pltpu.async_copy(fx_hbm.at[pl.ds(sbase, SPW)], bx, sem)
    cy = pltpu.async_copy(fy_hbm.at[pl.ds(sbase, SPW)], by, sem)
    cx.wait()
    cy.wait()

    cnt16[pl.ds(0, L)] = jnp.zeros((L,), jnp.int32)

    @pl.loop(0, SPW // L)
    def _p1(j):
        fl = _flat_vreg(bx, by, sbase, j)
        bfl[pl.ds(j * L, L)] = fl
        o = lax.shift_right_logical(fl, 14)
        cc, last = plsc.scan_count(o)
        cur = plsc.load_gather(cnt16, [o])
        plsc.store_scatter(cnt16, [o], cur + cc, mask=last)

    pltpu.sync_copy(cnt16, histS.at[pl.ds(sid * NS, NS)])
    pltpu.sync_copy(cnt16, histo_hbm.at[pl.ds((c * NS + sid) * NS, NS)])
    cd = pltpu.async_copy(d_hbm.at[pl.ds(sbase, SPW)], bdp, sem)
    c0 = pltpu.async_copy(i0_hbm.at[pl.ds(sbase, SPW)], bx, sem)
    c1 = pltpu.async_copy(i1_hbm.at[pl.ds(sbase, SPW)], by, sem)
    c2 = pltpu.async_copy(i2_hbm.at[pl.ds(sbase, SPW)], b2p, sem)
    cd.wait()
    c0.wait()
    c1.wait()
    c2.wait()
    plsc.subcore_barrier()

    pltpu.sync_copy(histS, bh)
    excl, tot, pre, padt = _bucket_layout(bh, sid)
    rstart = pl.multiple_of(_lane_scalar(excl, sid), L)
    rcnt_pad = pl.multiple_of(_lane_scalar(padt, sid), L)
    gapstart = rstart + _lane_scalar(tot, sid)

    padv[pl.ds(0, L)] = jnp.full((L,), -1, jnp.int32)
    iot = lax.iota(jnp.int32, L)
    pltpu.async_copy(padv, sfb.at[gapstart + iot], sem).wait()
    plsc.subcore_barrier()

    cnt16[pl.ds(0, L)] = excl + pre

    @pl.loop(0, SPW // L)
    def _p2(j):
        fl = bfl[pl.ds(j * L, L)]
        o = lax.shift_right_logical(fl, 14)
        cc, last = plsc.scan_count(o)
        cur = plsc.load_gather(cnt16, [o])
        didx[j // 8, pl.ds((j % 8) * L, L)] = cur + cc - 1
        plsc.store_scatter(cnt16, [o], cur + cc, mask=last)

    @pl.loop(0, SPW // 128)
    def _scat(r):
        off = r * 128
        sf = pltpu.async_copy(bfl.at[pl.ds(off, 128)], sfb.at[didx.at[r]], sem)
        sd = pltpu.async_copy(bdp.at[pl.ds(off, 128)], sdb.at[didx.at[r]], sem)
        s0 = pltpu.async_copy(bx.at[pl.ds(off, 128)], s0b.at[didx.at[r]], sem)
        s1 = pltpu.async_copy(by.at[pl.ds(off, 128)], s1b.at[didx.at[r]], sem)
        s2 = pltpu.async_copy(b2p.at[pl.ds(off, 128)], s2b.at[didx.at[r]], sem)
        sf.wait()
        sd.wait()
        s0.wait()
        s1.wait()
        s2.wait()

    plsc.subcore_barrier()

    big = jnp.full((L,), 1e30, jnp.float32)

    @pl.loop(0, TPB // L)
    def _init(i):
        zbuf[pl.ds(i * L, L)] = big

    nch = (rcnt_pad + CH - 1) // CH

    @pl.loop(0, nch)
    def _pass1(ci):
        cbase = pl.multiple_of(rstart + ci * CH, L)
        cf = pltpu.async_copy(sfb.at[pl.ds(cbase, CH)], bf, sem)
        cdd = pltpu.async_copy(sdb.at[pl.ds(cbase, CH)], bdp, sem)
        cf.wait()
        cdd.wait()
        done = ci * CH

        @pl.loop(0, CH // L)
        def _vreg(j):
            off = j * L
            fl = bf[pl.ds(off, L)]
            dd = bdp[pl.ds(off, L)]
            valid = ((done + off + iot) < rcnt_pad) & (fl >= 0)
            ridx = (fl - tbase) & (TPB - 1)

            def _body(_):
                cur = plsc.load_gather(zbuf, [ridx], mask=valid)
                pend = valid & (dd < cur)
                plsc.store_scatter(zbuf, [ridx], dd, mask=pend)
                cur2 = plsc.load_gather(zbuf, [ridx], mask=valid)
                return _any_f32(valid & (dd < cur2))

            lax.while_loop(lambda t: t > 0.0, _body, _any_f32(valid))

    pltpu.sync_copy(zbuf, pz_hbm.at[pl.ds(c * N + tbase, TPB)])

    half = STW // 2

    @pl.loop(0, 2)
    def _dump(hh):
        strip = sid * STW + hh * half
        hb = c * NBIN + strip
        pltpu.sync_copy(sfb.at[pl.ds(strip, half)], bf.at[pl.ds(0, half)])
        pltpu.sync_copy(bf.at[pl.ds(0, half)], fb_hbm.at[pl.ds(hb, half)])

        def _one(spm, hbm):
            pltpu.sync_copy(spm.at[pl.ds(strip, half)], bdp.at[pl.ds(0, half)])
            pltpu.sync_copy(bdp.at[pl.ds(0, half)], hbm.at[pl.ds(hb, half)])

        _one(sdb, db_hbm)
        _one(s0b, v0_hbm)
        _one(s1b, v1_hbm)
        _one(s2b, v2_hbm)


def _acc_body(histo_hbm, pz_hbm, fb_hbm, db_hbm, v0_hbm, v1_hbm, v2_hbm,
              sr_hbm, pa0_hbm, pa1_hbm, pa2_hbm, pcn_hbm,
              zbufG, acc0, acc1, acc2, cnt, bh, bf, bd, b0, b1, b2, bsr, sem):
    c = lax.axis_index("c")
    sid = lax.axis_index("s")
    tbase = sid * TPB

    pltpu.sync_copy(sr_hbm, bsr)
    srv = bsr[...]

    @pl.loop(0, 2)
    def _zmerge(hhalf):
        zoff = hhalf * CH
        pltpu.sync_copy(pz_hbm.at[pl.ds(tbase + zoff, CH)],
                        zbufG.at[pl.ds(zoff, CH)])
        pltpu.sync_copy(pz_hbm.at[pl.ds(N + tbase + zoff, CH)], bd)

        @pl.loop(0, CH // L)
        def _mn(i):
            off = zoff + i * L
            zbufG[pl.ds(off, L)] = jnp.minimum(zbufG[pl.ds(off, L)],
                                               bd[pl.ds(i * L, L)])

    zero = jnp.zeros((L,), jnp.float32)

    @pl.loop(0, TPB // L)
    def _init(i):
        off = i * L
        acc0[pl.ds(off, L)] = zero
        acc1[pl.ds(off, L)] = zero
        acc2[pl.ds(off, L)] = zero
        cnt[pl.ds(off, L)] = zero

    pltpu.sync_copy(histo_hbm.at[pl.ds(c * NS * NS, NS * NS)], bh)
    excl, _, _, padt = _bucket_layout(bh, sid)
    rstart = pl.multiple_of(_lane_scalar(excl, sid), L)
    rcnt_pad = pl.multiple_of(_lane_scalar(padt, sid), L)
    nch = (rcnt_pad + CH - 1) // CH
    iot = lax.iota(jnp.int32, L)

    @pl.loop(0, nch)
    def _pass2(ci):
        cbase = pl.multiple_of(c * NBIN + rstart + ci * CH, L)
        cf = pltpu.async_copy(fb_hbm.at[pl.ds(cbase, CH)], bf, sem)
        cd = pltpu.async_copy(db_hbm.at[pl.ds(cbase, CH)], bd, sem)
        c0 = pltpu.async_copy(v0_hbm.at[pl.ds(cbase, CH)], b0, sem)
        c1 = pltpu.async_copy(v1_hbm.at[pl.ds(cbase, CH)], b1, sem)
        c2 = pltpu.async_copy(v2_hbm.at[pl.ds(cbase, CH)], b2, sem)
        cf.wait()
        cd.wait()
        c0.wait()
        c1.wait()
        c2.wait()
        done = ci * CH

        @pl.loop(0, CH // L)
        def _vreg(j):
            off = j * L
            fl = bf[pl.ds(off, L)]
            dd = bd[pl.ds(off, L)]
            valid = ((done + off + iot) < rcnt_pad) & (fl >= 0)
            ridx = (fl - tbase) & (TPB - 1)
            zm = plsc.load_gather(zbufG, [ridx], mask=valid)
            ok = valid & (dd <= zm + srv)
            one = jnp.where(ok, jnp.full((L,), 1.0, jnp.float32), zero)
            plsc.addupdate_scatter(cnt, [ridx], one, mask=ok)
            plsc.addupdate_scatter(acc0, [ridx], b0[pl.ds(off, L)], mask=ok)
            plsc.addupdate_scatter(acc1, [ridx], b1[pl.ds(off, L)], mask=ok)
            plsc.addupdate_scatter(acc2, [ridx], b2[pl.ds(off, L)], mask=ok)

    base = c * N + tbase
    pltpu.sync_copy(acc0, pa0_hbm.at[pl.ds(base, TPB)])
    pltpu.sync_copy(acc1, pa1_hbm.at[pl.ds(base, TPB)])
    pltpu.sync_copy(acc2, pa2_hbm.at[pl.ds(base, TPB)])
    pltpu.sync_copy(cnt, pcn_hbm.at[pl.ds(base, TPB)])


def _merge_body(pa0_hbm, pa1_hbm, pa2_hbm, pcn_hbm,
                o0_hbm, o1_hbm, o2_hbm, ba, bb, binv, sem):
    wid = lax.axis_index("s") * NC + lax.axis_index("c")
    base = wid * SPW

    ca = pltpu.async_copy(pcn_hbm.at[pl.ds(base, SPW)], ba, sem)
    cb = pltpu.async_copy(pcn_hbm.at[pl.ds(N + base, SPW)], bb, sem)
    ca.wait()
    cb.wait()

    @pl.loop(0, SPW // L)
    def _inv(i):
        off = i * L
        tot = ba[pl.ds(off, L)] + bb[pl.ds(off, L)]
        binv[pl.ds(off, L)] = 1.0 / jnp.maximum(tot, 1.0)

    def _one(src_hbm, dst_hbm):
        ua = pltpu.async_copy(src_hbm.at[pl.ds(base, SPW)], ba, sem)
        ub = pltpu.async_copy(src_hbm.at[pl.ds(N + base, SPW)], bb, sem)
        ua.wait()
        ub.wait()

        @pl.loop(0, SPW // L)
        def _avg(i):
            off = i * L
            ba[pl.ds(off, L)] = ((ba[pl.ds(off, L)] + bb[pl.ds(off, L)])
                                 * binv[pl.ds(off, L)])

        pltpu.sync_copy(ba, dst_hbm.at[pl.ds(base, SPW)])

    _one(pa0_hbm, o0_hbm)
    _one(pa1_hbm, o1_hbm)
    _one(pa2_hbm, o2_hbm)


@jax.jit
def _run(fx, fy, d, i0, i1, i2, srv):
    mesh = plsc.VectorSubcoreMesh(core_axis_name="c", subcore_axis_name="s")
    params = pltpu.CompilerParams(needs_layout_passes=False)
    f32 = jnp.float32
    i32 = jnp.int32

    pz, histo, fb, db, v0, v1, v2 = pl.kernel(
        _bin_body,
        out_type=(
            jax.ShapeDtypeStruct((NC * N,), f32),
            jax.ShapeDtypeStruct((NC * NS * NS,), i32),
            jax.ShapeDtypeStruct((NC * NBIN,), i32),
            jax.ShapeDtypeStruct((NC * NBIN,), f32),
            jax.ShapeDtypeStruct((NC * NBIN,), f32),
            jax.ShapeDtypeStruct((NC * NBIN,), f32),
            jax.ShapeDtypeStruct((NC * NBIN,), f32),
        ),
        mesh=mesh,
        compiler_params=params,
        scratch_types=[
            pltpu.VMEM((SPW,), i32),
            pltpu.VMEM((SPW,), f32),
            pltpu.VMEM((SPW,), f32),
            pltpu.VMEM((SPW,), f32),
            pltpu.VMEM((SPW,), f32),
            pltpu.VMEM((SPW // 128, 128), i32),
            pltpu.VMEM((CH,), i32),
            pltpu.VMEM((TPB,), f32),
            pltpu.VMEM((NS * NS,), i32),
            pltpu.VMEM((L,), i32),
            pltpu.VMEM((L,), i32),
            pltpu.VMEM_SHARED((NBIN,), i32),
            pltpu.VMEM_SHARED((NBIN,), f32),
            pltpu.VMEM_SHARED((NBIN,), f32),
            pltpu.VMEM_SHARED((NBIN,), f32),
            pltpu.VMEM_SHARED((NBIN,), f32),
            pltpu.VMEM_SHARED((NS * NS,), i32),
            pltpu.SemaphoreType.DMA,
        ],
    )(fx, fy, d, i0, i1, i2)

    pa0, pa1, pa2, pcn = pl.kernel(
        _acc_body,
        out_type=(
            jax.ShapeDtypeStruct((NC * N,), f32),
            jax.ShapeDtypeStruct((NC * N,), f32),
            jax.ShapeDtypeStruct((NC * N,), f32),
            jax.ShapeDtypeStruct((NC * N,), f32),
        ),
        mesh=mesh,
        compiler_params=params,
        scratch_types=[
            pltpu.VMEM((TPB,), f32),
            pltpu.VMEM((TPB,), f32),
            pltpu.VMEM((TPB,), f32),
            pltpu.VMEM((TPB,), f32),
            pltpu.VMEM((TPB,), f32),
            pltpu.VMEM((NS * NS,), i32),
            pltpu.VMEM((CH,), i32),
            pltpu.VMEM((CH,), f32),
            pltpu.VMEM((CH,), f32),
            pltpu.VMEM((CH,), f32),
            pltpu.VMEM((CH,), f32),
            pltpu.VMEM((L,), f32),
            pltpu.SemaphoreType.DMA,
        ],
    )(histo, pz, fb, db, v0, v1, v2, srv)

    o0, o1, o2 = pl.kernel(
        _merge_body,
        out_type=(
            jax.ShapeDtypeStruct((N,), f32),
            jax.ShapeDtypeStruct((N,), f32),
            jax.ShapeDtypeStruct((N,), f32),
        ),
        mesh=mesh,
        compiler_params=params,
        scratch_types=[
            pltpu.VMEM((SPW,), f32),
            pltpu.VMEM((SPW,), f32),
            pltpu.VMEM((SPW,), f32),
            pltpu.SemaphoreType.DMA,
        ],
    )(pa0, pa1, pa2, pcn)

    out = jnp.stack([o0, o1, o2], axis=-1)
    return out.reshape(H, W, 3)


def kernel(img, flow, depth, same_range):
    fx = flow[0, :, :, 0].reshape(-1)
    fy = flow[0, :, :, 1].reshape(-1)
    d = depth.reshape(-1)
    i0 = img[:, :, 0].reshape(-1)
    i1 = img[:, :, 1].reshape(-1)
    i2 = img[:, :, 2].reshape(-1)
    srv = jnp.full((L,), same_range, jnp.float32)
    return _run(fx, fy, d, i0, i1, i2, srv)


# speedup vs baseline: 13.9188x; 1.0719x over previous
import jax
import jax.numpy as jnp
from jax import lax
from jax.experimental import pallas as pl
from jax.experimental.pallas import tpu as pltpu
from jax.experimental.pallas import tpu_sc as plsc

H = 512
W = 512
N = H * W
NC = 2
NS = 16
L = 16
NW = NC * NS
SPW = N // NW
TPB = N // NS
CH = 8192
NSRC = N // NC
NBIN = NSRC + NS * L + CH
STW = NBIN // NS


def _round_half_even_nonneg(x):
    n = x.astype(jnp.int32)
    f = x - n.astype(jnp.float32)
    half = jnp.full((L,), 0.5, jnp.float32)
    up = (f > half) | ((f == half) & ((n & 1) == 1))
    return jnp.where(up, n + 1, n)


def _flat_vreg(bx, by, base, j):
    off = j * L
    lin = base + off + lax.iota(jnp.int32, L)
    xi = lin & (W - 1)
    yi = lax.shift_right_logical(lin, 9)
    px = xi.astype(jnp.float32) + bx[pl.ds(off, L)]
    py = yi.astype(jnp.float32) + by[pl.ds(off, L)]
    px = jnp.minimum(jnp.maximum(px, 0.0), float(W - 1))
    py = jnp.minimum(jnp.maximum(py, 0.0), float(H - 1))
    tx = _round_half_even_nonneg(px)
    ty = _round_half_even_nonneg(py)
    return lax.shift_left(ty, 9) | tx


def _any_f32(p):
    return jnp.sum(jnp.where(p, jnp.full((L,), 1.0, jnp.float32),
                             jnp.zeros((L,), jnp.float32)))


def _bucket_layout(bh, sid):
    zero_i = jnp.zeros((L,), jnp.int32)

    def _acc(wi, carry):
        t, p = carry
        row = bh[pl.ds(wi * NS, L)]
        sel = jnp.full((L,), jnp.where(wi < sid, 1, 0), jnp.int32)
        return (t + row, p + row * sel)

    tot, pre = lax.fori_loop(0, NS, _acc, (zero_i, zero_i))
    padt = (tot + (L - 1)) & ~(L - 1)
    excl = plsc.cumsum(padt) - padt
    return excl, tot, pre, padt


def _lane_scalar(vec, sid):
    iot = lax.iota(jnp.int32, L)
    lane = jnp.where(iot == jnp.full((L,), sid),
                     jnp.full((L,), 1, jnp.int32), jnp.zeros((L,), jnp.int32))
    return jnp.sum(vec * lane)


def _bin_body(fx_hbm, fy_hbm, d_hbm, i0_hbm, i1_hbm, i2_hbm,
              pz_hbm, histo_hbm, fb_hbm, db_hbm, v0_hbm, v1_hbm, v2_hbm,
              bfl, bx, by, bdp, b2p, didx, bf, zbuf, bh, cnt16, padv,
              sfb, sdb, s0b, s1b, s2b, histS, sem):
    c = lax.axis_index("c")
    sid = lax.axis_index("s")
    wid = sid * NC + c
    sbase = wid * SPW
    tbase = sid * TPB

    cx = pltpu.async_copy(fx_hbm.at[pl.ds(sbase, SPW)], bx, sem)
    cy = pltpu.async_copy(fy_hbm.at[pl.ds(sbase, SPW)], by, sem)
    cx.wait()
    cy.wait()

    cnt16[pl.ds(0, L)] = jnp.zeros((L,), jnp.int32)

    @pl.loop(0, SPW // L)
    def _p1(j):
        fl = _flat_vreg(bx, by, sbase, j)
        bfl[pl.ds(j * L, L)] = fl
        o = lax.shift_right_logical(fl, 14)
        cc, last = plsc.scan_count(o)
        cur = plsc.load_gather(cnt16, [o])
        plsc.store_scatter(cnt16, [o], cur + cc, mask=last)

    pltpu.sync_copy(cnt16, histS.at[pl.ds(sid * NS, NS)])
    pltpu.sync_copy(cnt16, histo_hbm.at[pl.ds((c * NS + sid) * NS, NS)])
    cd = pltpu.async_copy(d_hbm.at[pl.ds(sbase, SPW)], bdp, sem)
    c0 = pltpu.async_copy(i0_hbm.at[pl.ds(sbase, SPW)], bx, sem)
    c1 = pltpu.async_copy(i1_hbm.at[pl.ds(sbase, SPW)], by, sem)
    c2 = pltpu.async_copy(i2_hbm.at[pl.ds(sbase, SPW)], b2p, sem)
    cd.wait()
    c0.wait()
    c1.wait()
    c2.wait()
    plsc.subcore_barrier()

    pltpu.sync_copy(histS, bh)
    excl, tot, pre, padt = _bucket_layout(bh, sid)
    rstart = pl.multiple_of(_lane_scalar(excl, sid), L)
    rcnt_pad = pl.multiple_of(_lane_scalar(padt, sid), L)
    gapstart = rstart + _lane_scalar(tot, sid)

    padv[pl.ds(0, L)] = jnp.full((L,), -1, jnp.int32)
    iot = lax.iota(jnp.int32, L)
    pltpu.async_copy(padv, sfb.at[gapstart + iot], sem).wait()
    plsc.subcore_barrier()

    cnt16[pl.ds(0, L)] = excl + pre

    @pl.loop(0, SPW // L)
    def _p2(j):
        fl = bfl[pl.ds(j * L, L)]
        o = lax.shift_right_logical(fl, 14)
        cc, last = plsc.scan_count(o)
        cur = plsc.load_gather(cnt16, [o])
        didx[j // 8, pl.ds((j % 8) * L, L)] = cur + cc - 1
        plsc.store_scatter(cnt16, [o], cur + cc, mask=last)

    @pl.loop(0, SPW // 128)
    def _scat(r):
        off = r * 128
        sf = pltpu.async_copy(bfl.at[pl.ds(off, 128)], sfb.at[didx.at[r]], sem)
        sd = pltpu.async_copy(bdp.at[pl.ds(off, 128)], sdb.at[didx.at[r]], sem)
        s0 = pltpu.async_copy(bx.at[pl.ds(off, 128)], s0b.at[didx.at[r]], sem)
        s1 = pltpu.async_copy(by.at[pl.ds(off, 128)], s1b.at[didx.at[r]], sem)
        s2 = pltpu.async_copy(b2p.at[pl.ds(off, 128)], s2b.at[didx.at[r]], sem)
        sf.wait()
        sd.wait()
        s0.wait()
        s1.wait()
        s2.wait()

    plsc.subcore_barrier()

    big = jnp.full((L,), 1e30, jnp.float32)

    @pl.loop(0, TPB // L)
    def _init(i):
        zbuf[pl.ds(i * L, L)] = big

    nch = (rcnt_pad + CH - 1) // CH

    @pl.loop(0, nch)
    def _pass1(ci):
        cbase = pl.multiple_of(rstart + ci * CH, L)
        cf = pltpu.async_copy(sfb.at[pl.ds(cbase, CH)], bf, sem)
        cdd = pltpu.async_copy(sdb.at[pl.ds(cbase, CH)], bdp, sem)
        cf.wait()
        cdd.wait()
        done = ci * CH

        @pl.loop(0, CH // L)
        def _vreg(j):
            off = j * L
            fl = bf[pl.ds(off, L)]
            dd = bdp[pl.ds(off, L)]
            valid = ((done + off + iot) < rcnt_pad) & (fl >= 0)
            ridx = (fl - tbase) & (TPB - 1)

            def _body(_):
                cur = plsc.load_gather(zbuf, [ridx], mask=valid)
                pend = valid & (dd < cur)
                plsc.store_scatter(zbuf, [ridx], dd, mask=pend)
                cur2 = plsc.load_gather(zbuf, [ridx], mask=valid)
                return _any_f32(valid & (dd < cur2))

            lax.while_loop(lambda t: t > 0.0, _body, jnp.float32(1.0))

    pltpu.sync_copy(zbuf, pz_hbm.at[pl.ds(c * N + tbase, TPB)])

    half = STW // 2

    @pl.loop(0, 2)
    def _dump(hh):
        strip = sid * STW + hh * half
        hb = c * NBIN + strip
        stage = ((sfb, bf, fb_hbm), (sdb, bdp, db_hbm), (s0b, bx, v0_hbm),
                 (s1b, by, v1_hbm), (s2b, b2p, v2_hbm))
        ins = [pltpu.async_copy(spm.at[pl.ds(strip, half)],
                                vm.at[pl.ds(0, half)], sem)
               for spm, vm, _ in stage]
        for cp in ins:
            cp.wait()
        outs = [pltpu.async_copy(vm.at[pl.ds(0, half)],
                                 hbm.at[pl.ds(hb, half)], sem)
                for _, vm, hbm in stage]
        for cp in outs:
            cp.wait()


def _acc_body(histo_hbm, pz_hbm, fb_hbm, db_hbm, v0_hbm, v1_hbm, v2_hbm,
              sr_hbm, pa0_hbm, pa1_hbm, pa2_hbm, pcn_hbm,
              zbufG, acc0, acc1, acc2, cnt, bh, bf, bd, b0, b1, b2, bsr, sem):
    c = lax.axis_index("c")
    sid = lax.axis_index("s")
    tbase = sid * TPB

    pltpu.sync_copy(sr_hbm, bsr)
    srv = bsr[...]

    @pl.loop(0, 2)
    def _zmerge(hhalf):
        zoff = hhalf * CH
        pltpu.sync_copy(pz_hbm.at[pl.ds(tbase + zoff, CH)],
                        zbufG.at[pl.ds(zoff, CH)])
        pltpu.sync_copy(pz_hbm.at[pl.ds(N + tbase + zoff, CH)], bd)

        @pl.loop(0, CH // L)
        def _mn(i):
            off = zoff + i * L
            zbufG[pl.ds(off, L)] = jnp.minimum(zbufG[pl.ds(off, L)],
                                               bd[pl.ds(i * L, L)])

    zero = jnp.zeros((L,), jnp.float32)

    @pl.loop(0, TPB // L)
    def _init(i):
        off = i * L
        acc0[pl.ds(off, L)] = zero
        acc1[pl.ds(off, L)] = zero
        acc2[pl.ds(off, L)] = zero
        cnt[pl.ds(off, L)] = zero

    pltpu.sync_copy(histo_hbm.at[pl.ds(c * NS * NS, NS * NS)], bh)
    excl, _, _, padt = _bucket_layout(bh, sid)
    rstart = pl.multiple_of(_lane_scalar(excl, sid), L)
    rcnt_pad = pl.multiple_of(_lane_scalar(padt, sid), L)
    nch = (rcnt_pad + CH - 1) // CH
    iot = lax.iota(jnp.int32, L)

    @pl.loop(0, nch)
    def _pass2(ci):
        cbase = pl.multiple_of(c * NBIN + rstart + ci * CH, L)
        cf = pltpu.async_copy(fb_hbm.at[pl.ds(cbase, CH)], bf, sem)
        cd = pltpu.async_copy(db_hbm.at[pl.ds(cbase, CH)], bd, sem)
        c0 = pltpu.async_copy(v0_hbm.at[pl.ds(cbase, CH)], b0, sem)
        c1 = pltpu.async_copy(v1_hbm.at[pl.ds(cbase, CH)], b1, sem)
        c2 = pltpu.async_copy(v2_hbm.at[pl.ds(cbase, CH)], b2, sem)
        cf.wait()
        cd.wait()
        c0.wait()
        c1.wait()
        c2.wait()
        done = ci * CH

        @pl.loop(0, CH // L)
        def _vreg(j):
            off = j * L
            fl = bf[pl.ds(off, L)]
            dd = bd[pl.ds(off, L)]
            valid = ((done + off + iot) < rcnt_pad) & (fl >= 0)
            ridx = (fl - tbase) & (TPB - 1)
            zm = plsc.load_gather(zbufG, [ridx], mask=valid)
            ok = valid & (dd <= zm + srv)
            one = jnp.where(ok, jnp.full((L,), 1.0, jnp.float32), zero)
            plsc.addupdate_scatter(cnt, [ridx], one, mask=ok)
            plsc.addupdate_scatter(acc0, [ridx], b0[pl.ds(off, L)], mask=ok)
            plsc.addupdate_scatter(acc1, [ridx], b1[pl.ds(off, L)], mask=ok)
            plsc.addupdate_scatter(acc2, [ridx], b2[pl.ds(off, L)], mask=ok)

    base = c * N + tbase
    w0 = pltpu.async_copy(acc0, pa0_hbm.at[pl.ds(base, TPB)], sem)
    w1 = pltpu.async_copy(acc1, pa1_hbm.at[pl.ds(base, TPB)], sem)
    w2 = pltpu.async_copy(acc2, pa2_hbm.at[pl.ds(base, TPB)], sem)
    w3 = pltpu.async_copy(cnt, pcn_hbm.at[pl.ds(base, TPB)], sem)
    w0.wait()
    w1.wait()
    w2.wait()
    w3.wait()


def _merge_body(pa0_hbm, pa1_hbm, pa2_hbm, pcn_hbm,
                o0_hbm, o1_hbm, o2_hbm,
                ca_, cb_, a0, b0, a1, b1, a2, b2, binv, sem):
    wid = lax.axis_index("s") * NC + lax.axis_index("c")
    base = wid * SPW

    pairs = ((pcn_hbm, ca_, cb_), (pa0_hbm, a0, b0), (pa1_hbm, a1, b1),
             (pa2_hbm, a2, b2))
    ins = []
    for src, pa, pb in pairs:
        ins.append(pltpu.async_copy(src.at[pl.ds(base, SPW)], pa, sem))
        ins.append(pltpu.async_copy(src.at[pl.ds(N + base, SPW)], pb, sem))
    ins[0].wait()
    ins[1].wait()

    @pl.loop(0, SPW // L)
    def _inv(i):
        off = i * L
        tot = ca_[pl.ds(off, L)] + cb_[pl.ds(off, L)]
        binv[pl.ds(off, L)] = 1.0 / jnp.maximum(tot, 1.0)

    for cp in ins[2:]:
        cp.wait()

    outs = []
    for (_, pa, pb), dst in zip(pairs[1:], (o0_hbm, o1_hbm, o2_hbm)):
        @pl.loop(0, SPW // L)
        def _avg(i, pa=pa, pb=pb):
            off = i * L
            pa[pl.ds(off, L)] = ((pa[pl.ds(off, L)] + pb[pl.ds(off, L)])
                                 * binv[pl.ds(off, L)])

        outs.append(pltpu.async_copy(pa, dst.at[pl.ds(base, SPW)], sem))
    for cp in outs:
        cp.wait()


@jax.jit
def _run(fx, fy, d, i0, i1, i2, srv):
    mesh = plsc.VectorSubcoreMesh(core_axis_name="c", subcore_axis_name="s")
    params = pltpu.CompilerParams(needs_layout_passes=False)
    f32 = jnp.float32
    i32 = jnp.int32

    pz, histo, fb, db, v0, v1, v2 = pl.kernel(
        _bin_body,
        out_type=(
            jax.ShapeDtypeStruct((NC * N,), f32),
            jax.ShapeDtypeStruct((NC * NS * NS,), i32),
            jax.ShapeDtypeStruct((NC * NBIN,), i32),
            jax.ShapeDtypeStruct((NC * NBIN,), f32),
            jax.ShapeDtypeStruct((NC * NBIN,), f32),
            jax.ShapeDtypeStruct((NC * NBIN,), f32),
            jax.ShapeDtypeStruct((NC * NBIN,), f32),
        ),
        mesh=mesh,
        compiler_params=params,
        scratch_types=[
            pltpu.VMEM((SPW,), i32),
            pltpu.VMEM((SPW,), f32),
            pltpu.VMEM((SPW,), f32),
            pltpu.VMEM((SPW,), f32),
            pltpu.VMEM((SPW,), f32),
            pltpu.VMEM((SPW // 128, 128), i32),
            pltpu.VMEM((CH,), i32),
            pltpu.VMEM((TPB,), f32),
            pltpu.VMEM((NS * NS,), i32),
            pltpu.VMEM((L,), i32),
            pltpu.VMEM((L,), i32),
            pltpu.VMEM_SHARED((NBIN,), i32),
            pltpu.VMEM_SHARED((NBIN,), f32),
            pltpu.VMEM_SHARED((NBIN,), f32),
            pltpu.VMEM_SHARED((NBIN,), f32),
            pltpu.VMEM_SHARED((NBIN,), f32),
            pltpu.VMEM_SHARED((NS * NS,), i32),
            pltpu.SemaphoreType.DMA,
        ],
    )(fx, fy, d, i0, i1, i2)

    pa0, pa1, pa2, pcn = pl.kernel(
        _acc_body,
        out_type=(
            jax.ShapeDtypeStruct((NC * N,), f32),
            jax.ShapeDtypeStruct((NC * N,), f32),
            jax.ShapeDtypeStruct((NC * N,), f32),
            jax.ShapeDtypeStruct((NC * N,), f32),
        ),
        mesh=mesh,
        compiler_params=params,
        scratch_types=[
            pltpu.VMEM((TPB,), f32),
            pltpu.VMEM((TPB,), f32),
            pltpu.VMEM((TPB,), f32),
            pltpu.VMEM((TPB,), f32),
            pltpu.VMEM((TPB,), f32),
            pltpu.VMEM((NS * NS,), i32),
            pltpu.VMEM((CH,), i32),
            pltpu.VMEM((CH,), f32),
            pltpu.VMEM((CH,), f32),
            pltpu.VMEM((CH,), f32),
            pltpu.VMEM((CH,), f32),
            pltpu.VMEM((L,), f32),
            pltpu.SemaphoreType.DMA,
        ],
    )(histo, pz, fb, db, v0, v1, v2, srv)

    o0, o1, o2 = pl.kernel(
        _merge_body,
        out_type=(
            jax.ShapeDtypeStruct((N,), f32),
            jax.ShapeDtypeStruct((N,), f32),
            jax.ShapeDtypeStruct((N,), f32),
        ),
        mesh=mesh,
        compiler_params=params,
        scratch_types=[
            pltpu.VMEM((SPW,), f32),
            pltpu.VMEM((SPW,), f32),
            pltpu.VMEM((SPW,), f32),
            pltpu.VMEM((SPW,), f32),
            pltpu.VMEM((SPW,), f32),
            pltpu.VMEM((SPW,), f32),
            pltpu.VMEM((SPW,), f32),
            pltpu.VMEM((SPW,), f32),
            pltpu.VMEM((SPW,), f32),
            pltpu.SemaphoreType.DMA,
        ],
    )(pa0, pa1, pa2, pcn)

    out = jnp.stack([o0, o1, o2], axis=-1)
    return out.reshape(H, W, 3)


def kernel(img, flow, depth, same_range):
    fx = flow[0, :, :, 0].reshape(-1)
    fy = flow[0, :, :, 1].reshape(-1)
    d = depth.reshape(-1)
    i0 = img[:, :, 0].reshape(-1)
    i1 = img[:, :, 1].reshape(-1)
    i2 = img[:, :, 2].reshape(-1)
    srv = jnp.full((L,), same_range, jnp.float32)
    return _run(fx, fy, d, i0, i1, i2, srv)
